# baseline (speedup 1.0000x reference)
import jax
import jax.numpy as jnp
from jax import lax
from jax.experimental import pallas as pl
from jax.experimental.pallas import tpu as pltpu
from jax.experimental.pallas import tpu_sc as plsc

B, L, V, D, H, C = 16384, 50, 100000, 128, 50, 10

NC, NS = 2, 16
NW = NC * NS
ROWS_PER_W = B // NW
CHUNK_ROWS = 8
CHUNK_IDX = CHUNK_ROWS * L
NCHUNK = ROWS_PER_W // CHUNK_ROWS
GSPLIT = 5
GSIZE = CHUNK_IDX // GSPLIT
NLANE = 16
NVREG = D // NLANE


def _sc_embed_body(text_hbm, table_hbm, out_hbm, idx_v0, idx_v1, rows_v0,
                   rows_v1, obuf0, obuf1, gsem0, gsem1, isem0, isem1,
                   osem0, osem1):
    wid = lax.axis_index("s") * NC + lax.axis_index("c")
    base_idx = wid * (ROWS_PER_W * L)
    base_row = wid * ROWS_PER_W
    gsems = (gsem0, gsem1)
    isems = (isem0, isem1)
    osems = (osem0, osem1)
    idxs = (idx_v0, idx_v1)
    rows = (rows_v0, rows_v1)
    obufs = (obuf0, obuf1)

    def idx_src(c):
        return text_hbm.at[pl.ds(base_idx + c * CHUNK_IDX, CHUNK_IDX)]

    def idx_start(b, c):
        pltpu.async_copy(idx_src(c), idxs[b], isems[b])

    def idx_wait(b, c):
        pltpu.make_async_copy(idx_src(c), idxs[b], isems[b]).wait()

    def fire(b):
        for j in range(GSPLIT):
            pltpu.async_copy(
                table_hbm.at[idxs[b].at[pl.ds(j * GSIZE, GSIZE)]],
                rows[b].at[pl.ds(j * GSIZE, GSIZE)],
                gsems[b])

    def drain(b):
        for j in range(GSPLIT):
            pltpu.make_async_copy(
                table_hbm.at[idxs[b].at[pl.ds(j * GSIZE, GSIZE)]],
                rows[b].at[pl.ds(j * GSIZE, GSIZE)],
                gsems[b]).wait()

    def out_dst(c):
        return out_hbm.at[pl.ds(base_row + c * CHUNK_ROWS, CHUNK_ROWS)]

    def out_start(b, c):
        pltpu.async_copy(obufs[b], out_dst(c), osems[b])

    def out_wait(b, c):
        pltpu.make_async_copy(obufs[b], out_dst(c), osems[b]).wait()

    def reduce(b):
        for r in range(CHUNK_ROWS):
            def lbody(l, accs):
                row = r * L + l
                return tuple(accs[d] + rows[b][row, pl.ds(NLANE * d, NLANE)]
                             for d in range(NVREG))
            accs = lax.fori_loop(
                0, L, lbody,
                tuple(jnp.zeros((NLANE,), jnp.float32)
                      for _ in range(NVREG)))
            for d in range(NVREG):
                obufs[b][r, pl.ds(NLANE * d, NLANE)] = accs[d] * (1.0 / L)

    pltpu.sync_copy(idx_src(0), idxs[0])
    fire(0)
    pltpu.sync_copy(idx_src(1), idxs[1])
    out_start(0, 0)
    out_start(1, 1)

    def chunk_body(i, _):
        c0 = 2 * i
        fire(1)
        drain(0)
        c2 = jnp.minimum(c0 + 2, NCHUNK - 1)
        idx_start(0, c2)
        out_wait(0, jnp.maximum(c0 - 2, 0))
        reduce(0)
        out_start(0, c0)
        idx_wait(0, c2)
        fire(0)
        drain(1)
        c3 = jnp.minimum(c0 + 3, NCHUNK - 1)
        idx_start(1, c3)
        out_wait(1, jnp.maximum(c0 - 1, 1))
        reduce(1)
        out_start(1, c0 + 1)
        idx_wait(1, c3)
        return 0

    lax.fori_loop(0, NCHUNK // 2, chunk_body, 0)
    drain(0)
    out_wait(0, NCHUNK - 2)
    out_wait(1, NCHUNK - 1)


_sc_embed = pl.kernel(
    _sc_embed_body,
    out_type=jax.ShapeDtypeStruct((B, D), jnp.float32),
    mesh=plsc.VectorSubcoreMesh(core_axis_name="c", subcore_axis_name="s",
                                num_cores=NC, num_subcores=NS),
    scratch_types=[
        pltpu.VMEM((CHUNK_IDX,), jnp.int32),
        pltpu.VMEM((CHUNK_IDX,), jnp.int32),
        pltpu.VMEM((CHUNK_IDX, D), jnp.float32),
        pltpu.VMEM((CHUNK_IDX, D), jnp.float32),
        pltpu.VMEM((CHUNK_ROWS, D), jnp.float32),
        pltpu.VMEM((CHUNK_ROWS, D), jnp.float32),
    ] + [pltpu.SemaphoreType.DMA] * 6,
)


def _mlp_body(pooled_ref, ly_ref, W1a_ref, W1b_ref, b1_ref, W2_ref, b2_ref,
              W3_ref, b3_ref, out_ref):
    x = pooled_ref[...]
    h = jnp.dot(x, W1a_ref[...], preferred_element_type=jnp.float32)
    h += jnp.dot(ly_ref[...], W1b_ref[...], preferred_element_type=jnp.float32)
    h = jnp.maximum(h + b1_ref[...], 0.0)
    h = jnp.maximum(
        jnp.dot(h, W2_ref[...], preferred_element_type=jnp.float32)
        + b2_ref[...], 0.0)
    out_ref[...] = (jnp.dot(h, W3_ref[...], preferred_element_type=jnp.float32)
                    + b3_ref[...]).T


BM = 1024


def _mlp(pooled, ly, W1a, W1b, b1, W2, b2, W3, b3):
    grid = (B // BM,)
    return pl.pallas_call(
        _mlp_body,
        grid=grid,
        in_specs=[
            pl.BlockSpec((BM, D), lambda i: (i, 0)),
            pl.BlockSpec((BM, 2), lambda i: (i, 0)),
            pl.BlockSpec((D, H), lambda i: (0, 0)),
            pl.BlockSpec((2, H), lambda i: (0, 0)),
            pl.BlockSpec((1, H), lambda i: (0, 0)),
            pl.BlockSpec((H, H), lambda i: (0, 0)),
            pl.BlockSpec((1, H), lambda i: (0, 0)),
            pl.BlockSpec((H, C), lambda i: (0, 0)),
            pl.BlockSpec((1, C), lambda i: (0, 0)),
        ],
        out_specs=pl.BlockSpec((C, BM), lambda i: (0, i)),
        out_shape=jax.ShapeDtypeStruct((C, B), jnp.float32),
    )(pooled, ly, W1a, W1b, b1, W2, b2, W3, b3)


@jax.jit
def kernel(text, text_len, year, table, W1, b1, W2, b2, W3, b3):
    text_flat = text.astype(jnp.int32).reshape(-1)
    pooled = _sc_embed(text_flat, table)
    ly = jnp.stack([text_len.astype(jnp.float32),
                    year.astype(jnp.float32)], axis=1)
    W1a = W1[:D]
    W1b = W1[D:]
    out_t = _mlp(pooled, ly, W1a, W1b, b1.reshape(1, H), W2, b2.reshape(1, H),
                 W3, b3.reshape(1, C))
    return out_t.T


# speedup vs baseline: 1.0283x; 1.0283x over previous
import jax
import jax.numpy as jnp
from jax import lax
from jax.experimental import pallas as pl
from jax.experimental.pallas import tpu as pltpu
from jax.experimental.pallas import tpu_sc as plsc

B, L, V, D, H, C = 16384, 50, 100000, 128, 50, 10

NC, NS = 2, 16
NW = NC * NS
ROWS_PER_W = B // NW
CHUNK_ROWS = 8
CHUNK_IDX = CHUNK_ROWS * L
NCHUNK = ROWS_PER_W // CHUNK_ROWS
GSPLIT = 5
GSIZE = CHUNK_IDX // GSPLIT
NLANE = 16
NVREG = D // NLANE


def _sc_embed_body(text_hbm, table_hbm, out_hbm, idx_v0, idx_v1, rows_v0,
                   rows_v1, obuf0, obuf1, gsem0, gsem1, isem0, isem1,
                   osem0, osem1):
    wid = lax.axis_index("s") * NC + lax.axis_index("c")
    base_idx = wid * (ROWS_PER_W * L)
    base_row = wid * ROWS_PER_W
    gsems = (gsem0, gsem1)
    isems = (isem0, isem1)
    osems = (osem0, osem1)
    idxs = (idx_v0, idx_v1)
    rows = (rows_v0, rows_v1)
    obufs = (obuf0, obuf1)

    def idx_src(c):
        return text_hbm.at[pl.ds(base_idx + c * CHUNK_IDX, CHUNK_IDX)]

    def idx_start(b, c):
        pltpu.async_copy(idx_src(c), idxs[b], isems[b])

    def idx_wait(b, c):
        pltpu.make_async_copy(idx_src(c), idxs[b], isems[b]).wait()

    def fire(b):
        for j in range(GSPLIT):
            pltpu.async_copy(
                table_hbm.at[idxs[b].at[pl.ds(j * GSIZE, GSIZE)]],
                rows[b].at[pl.ds(j * GSIZE, GSIZE)],
                gsems[b])

    def drain(b):
        for j in range(GSPLIT):
            pltpu.make_async_copy(
                table_hbm.at[idxs[b].at[pl.ds(j * GSIZE, GSIZE)]],
                rows[b].at[pl.ds(j * GSIZE, GSIZE)],
                gsems[b]).wait()

    def out_dst(c):
        return out_hbm.at[pl.ds(base_row + c * CHUNK_ROWS, CHUNK_ROWS)]

    def out_start(b, c):
        pltpu.async_copy(obufs[b], out_dst(c), osems[b])

    def out_wait(b, c):
        pltpu.make_async_copy(obufs[b], out_dst(c), osems[b]).wait()

    def reduce(b):
        for r in range(CHUNK_ROWS):
            def lbody(l, accs):
                row = r * L + l
                return tuple(accs[d] + rows[b][row, pl.ds(NLANE * d, NLANE)]
                             for d in range(NVREG))
            accs = lax.fori_loop(
                0, L, lbody,
                tuple(jnp.zeros((NLANE,), jnp.float32)
                      for _ in range(NVREG)))
            for d in range(NVREG):
                obufs[b][r, pl.ds(NLANE * d, NLANE)] = accs[d] * (1.0 / L)

    pltpu.sync_copy(idx_src(0), idxs[0])
    fire(0)
    pltpu.sync_copy(idx_src(1), idxs[1])
    out_start(0, 0)
    out_start(1, 1)

    def chunk_body(i, _):
        c0 = 2 * i
        fire(1)
        drain(0)
        c2 = jnp.minimum(c0 + 2, NCHUNK - 1)
        idx_start(0, c2)
        out_wait(0, jnp.maximum(c0 - 2, 0))
        reduce(0)
        out_start(0, c0)
        idx_wait(0, c2)
        fire(0)
        drain(1)
        c3 = jnp.minimum(c0 + 3, NCHUNK - 1)
        idx_start(1, c3)
        out_wait(1, jnp.maximum(c0 - 1, 1))
        reduce(1)
        out_start(1, c0 + 1)
        idx_wait(1, c3)
        return 0

    lax.fori_loop(0, NCHUNK // 2, chunk_body, 0)
    drain(0)
    out_wait(0, NCHUNK - 2)
    out_wait(1, NCHUNK - 1)


_sc_embed = pl.kernel(
    _sc_embed_body,
    out_type=jax.ShapeDtypeStruct((B, D), jnp.float32),
    mesh=plsc.VectorSubcoreMesh(core_axis_name="c", subcore_axis_name="s",
                                num_cores=NC, num_subcores=NS),
    scratch_types=[
        pltpu.VMEM((CHUNK_IDX,), jnp.int32),
        pltpu.VMEM((CHUNK_IDX,), jnp.int32),
        pltpu.VMEM((CHUNK_IDX, D), jnp.float32),
        pltpu.VMEM((CHUNK_IDX, D), jnp.float32),
        pltpu.VMEM((CHUNK_ROWS, D), jnp.float32),
        pltpu.VMEM((CHUNK_ROWS, D), jnp.float32),
    ] + [pltpu.SemaphoreType.DMA] * 6,
)


def _mlp_body(pooled_ref, ly_ref, W1a_ref, W1b_ref, b1_ref, W2_ref, b2_ref,
              W3_ref, b3_ref, out_ref):
    x = pooled_ref[...]
    h = jnp.dot(x, W1a_ref[...], preferred_element_type=jnp.float32)
    h += jnp.dot(ly_ref[...], W1b_ref[...], preferred_element_type=jnp.float32)
    h = jnp.maximum(h + b1_ref[...], 0.0)
    h = jnp.maximum(
        jnp.dot(h, W2_ref[...], preferred_element_type=jnp.float32)
        + b2_ref[...], 0.0)
    out_ref[...] = (jnp.dot(h, W3_ref[...], preferred_element_type=jnp.float32)
                    + b3_ref[...]).T


BM = 8192


def _mlp(pooled, ly, W1a, W1b, b1, W2, b2, W3, b3):
    grid = (B // BM,)
    return pl.pallas_call(
        _mlp_body,
        grid=grid,
        in_specs=[
            pl.BlockSpec((BM, D), lambda i: (i, 0)),
            pl.BlockSpec((BM, 2), lambda i: (i, 0)),
            pl.BlockSpec((D, H), lambda i: (0, 0)),
            pl.BlockSpec((2, H), lambda i: (0, 0)),
            pl.BlockSpec((1, H), lambda i: (0, 0)),
            pl.BlockSpec((H, H), lambda i: (0, 0)),
            pl.BlockSpec((1, H), lambda i: (0, 0)),
            pl.BlockSpec((H, C), lambda i: (0, 0)),
            pl.BlockSpec((1, C), lambda i: (0, 0)),
        ],
        out_specs=pl.BlockSpec((C, BM), lambda i: (0, i)),
        out_shape=jax.ShapeDtypeStruct((C, B), jnp.float32),
    )(pooled, ly, W1a, W1b, b1, W2, b2, W3, b3)


@jax.jit
def kernel(text, text_len, year, table, W1, b1, W2, b2, W3, b3):
    text_flat = text.astype(jnp.int32).reshape(-1)
    pooled = _sc_embed(text_flat, table)
    ly = jnp.stack([text_len.astype(jnp.float32),
                    year.astype(jnp.float32)], axis=1)
    W1a = W1[:D]
    W1b = W1[D:]
    out_t = _mlp(pooled, ly, W1a, W1b, b1.reshape(1, H), W2, b2.reshape(1, H),
                 W3, b3.reshape(1, C))
    return out_t.T
